# Initial kernel scaffold; baseline (speedup 1.0000x reference)
import jax
import jax.numpy as jnp
from jax.experimental import pallas as pl


def kernel(x0, x1, x2, up0_index, up0_shared, up1_index, up1_shared, down1_index, down1_shared, down2_index, down2_shared, batch0, batch1, batch2, params):
    raise NotImplementedError("write your pallas kernel here")


# speedup vs baseline: 1.6274x; 1.6274x over previous
import functools

import jax
import jax.numpy as jnp
from jax import lax
from jax.experimental import pallas as pl
from jax.experimental.pallas import tpu as pltpu
from jax.experimental.pallas import tpu_sc as plsc

F = 128
NB = 64
NCLS = 10
CH = 6144
K = 128
SEGP = 128
NEG = -1e30

_f32 = jnp.float32
_i32 = jnp.int32


def _rup(n, m):
    return (n + m - 1) // m * m


def _pad_rows(x, np_):
    n = x.shape[0]
    if n == np_:
        return x
    return jnp.pad(x, ((0, np_ - n), (0, 0)))


def _mm_multi_body(nj, x_ref, w_ref, *out_refs):
    x = x_ref[...]
    for j in range(nj):
        out_refs[j][...] = jnp.dot(x, w_ref[j], preferred_element_type=_f32)


def _mm_multi(x, ws):
    nj = len(ws)
    npad = x.shape[0]
    br = 1024
    w = jnp.stack(ws)
    return pl.pallas_call(
        functools.partial(_mm_multi_body, nj),
        grid=(npad // br,),
        in_specs=[
            pl.BlockSpec((br, F), lambda i: (i, 0)),
            pl.BlockSpec((nj, F, F), lambda i: (0, 0, 0)),
        ],
        out_specs=[pl.BlockSpec((br, F), lambda i: (i, 0))] * nj,
        out_shape=[jax.ShapeDtypeStruct((npad, F), _f32)] * nj,
    )(x, w)


def _upd_body(y_ref, w_ref, v_ref, o_ref):
    y = y_ref[...]
    h = jnp.maximum(jnp.dot(y, w_ref[0], preferred_element_type=_f32)
                    + v_ref[0, :][None, :], 0.0)
    h = jnp.maximum(jnp.dot(h, w_ref[1], preferred_element_type=_f32)
                    + v_ref[1, :][None, :], 0.0)
    o_ref[...] = h * v_ref[2, :][None, :] + v_ref[3, :][None, :]


def _upd_mlp(y, w1, b1, w2, b2, g, be):
    npad = y.shape[0]
    br = 1024
    w = jnp.stack([w1, w2])
    v = jnp.stack([b1, b2, g, be, b1, b1, b1, b1])
    return pl.pallas_call(
        _upd_body,
        grid=(npad // br,),
        in_specs=[
            pl.BlockSpec((br, F), lambda i: (i, 0)),
            pl.BlockSpec((2, F, F), lambda i: (0, 0, 0)),
            pl.BlockSpec((8, F), lambda i: (0, 0)),
        ],
        out_specs=pl.BlockSpec((br, F), lambda i: (i, 0)),
        out_shape=jax.ShapeDtypeStruct((npad, F), _f32),
    )(y, w, v)


def _pool_body(bidx_ref, x_ref, s_ref, c_ref):
    i = pl.program_id(0)
    ids = bidx_ref[...].reshape(1, -1)
    seg = lax.broadcasted_iota(_i32, (SEGP, ids.shape[1]), 0)
    oh = (seg == ids).astype(_f32)
    ps = jnp.dot(oh, x_ref[...], preferred_element_type=_f32)
    pc = jnp.broadcast_to(jnp.sum(oh, axis=1)[:, None], (SEGP, F))

    @pl.when(i == 0)
    def _():
        s_ref[...] = jnp.zeros_like(s_ref)
        c_ref[...] = jnp.zeros_like(c_ref)

    s_ref[...] += ps
    c_ref[...] += pc


def _pool(x, bidx3d):
    npad = x.shape[0]
    br = 1024
    nblk = npad // br
    return pl.pallas_call(
        _pool_body,
        grid=(nblk,),
        in_specs=[
            pl.BlockSpec((1, 1, br), lambda i: (i, 0, 0)),
            pl.BlockSpec((br, F), lambda i: (i, 0)),
        ],
        out_specs=[
            pl.BlockSpec((SEGP, F), lambda i: (0, 0)),
            pl.BlockSpec((SEGP, F), lambda i: (0, 0)),
        ],
        out_shape=[jax.ShapeDtypeStruct((SEGP, F), _f32)] * 2,
    )(bidx3d, x)


def _head_body(s0, c0, s1, c1, s2, c2, w_ref, v_ref, o_ref):
    p = (s0[...] / jnp.maximum(c0[...], 1.0)
         + s1[...] / jnp.maximum(c1[...], 1.0)
         + s2[...] / jnp.maximum(c2[...], 1.0))
    h = jnp.maximum(jnp.dot(p, w_ref[0], preferred_element_type=_f32)
                    + v_ref[0, :][None, :], 0.0)
    lg = jnp.dot(h, w_ref[1], preferred_element_type=_f32) + v_ref[1, :][None, :]
    m = jnp.max(lg, axis=1, keepdims=True)
    lse = m + jnp.log(jnp.sum(jnp.exp(lg - m), axis=1, keepdims=True))
    o_ref[...] = lg - lse


def _head(s0, c0, s1, c1, s2, c2, w1, b1, w2, b2):
    w2p = jnp.pad(w2, ((0, 0), (0, F - NCLS)))
    b2p = jnp.pad(b2, (0, F - NCLS), constant_values=NEG)
    w = jnp.stack([w1, w2p])
    v = jnp.stack([b1, b2p, b1, b1, b1, b1, b1, b1])
    spec = pl.BlockSpec((SEGP, F), lambda: (0, 0))
    return pl.pallas_call(
        _head_body,
        in_specs=[spec] * 6 + [pl.BlockSpec((2, F, F), lambda: (0, 0, 0)),
                               pl.BlockSpec((8, F), lambda: (0, 0))],
        out_specs=spec,
        out_shape=jax.ShapeDtypeStruct((SEGP, F), _f32),
    )(s0, c0, s1, c1, s2, c2, w, v)


_N0, _N1, _N2 = 10000, 80000, 20000
_NP0, _NP1, _NP2 = _rup(_N0, CH), _rup(_N1, CH), _rup(_N2, CH)
_EU0, _EU1, _ED1, _ED2 = 160000, 120000, 160000, 60000
_EP = [_rup(e, 256) for e in (_EU0, _EU1, _ED1, _ED2)]
_SHARES = [e // 16 for e in _EP]
_SHARE_MAX = max(_SHARES)
_QCAP = _SHARE_MAX + 16 + K

_TASKS = []
for _t, _npad in ((0, _NP0), (1, _NP1), (2, _NP2)):
    for _ci, _lo in enumerate(range(0, _npad, CH)):
        _TASKS.append((_t, _lo, _ci % 2))


_TABLE_LISTS = {
    0: [(0, 3, 4, True)],
    1: [(1, 5, 6, True), (2, 7, 8, False)],
    2: [(3, 9, 10, False)],
}


def _sc_body(*refs):
    (xp0, xp1, xp2,
     t0u, t1ub, t1ut, t2ub, t1dt, t0db, t2dt, t1db,
     e_src0, e_sh0, e_dst0, e_src1, e_sh1, e_dst1,
     e_src2, e_sh2, e_dst2, e_src3, e_sh3, e_dst3,
     upb_h, upg_h, upbe_h, dnb_h, dng_h, dnbe_h,
     out0, out1, out2,
     dshare, sshare, hshare, qpos, blk_src, blk_sh, blk_dst,
     buf_a, buf_b, pvm, acc, sem_a, sem_b) = refs

    cid = lax.axis_index("c")
    tid = lax.axis_index("s")

    xs = (xp0, xp1, xp2)
    outs = (out0, out1, out2)
    tabs = (None, None, None, t0u, t1ub, t1ut, t2ub, t1dt, t0db, t2dt, t1db)
    elists = ((e_src0, e_sh0, e_dst0), (e_src1, e_sh1, e_dst1),
              (e_src2, e_sh2, e_dst2), (e_src3, e_sh3, e_dst3))

    for j, ph in enumerate((upb_h, upg_h, upbe_h, dnb_h, dng_h, dnbe_h)):
        pltpu.sync_copy(ph, pvm.at[j])

    rows_pt = CH // 16

    def run_list(lid, srcT, shT, is_up, lo):
        share = _SHARES[lid]
        e_src, e_sh, e_dst = elists[lid]
        base = tid * share
        pltpu.sync_copy(e_dst.at[pl.ds(base, share)], dshare.at[pl.ds(0, share)])
        pltpu.sync_copy(e_src.at[pl.ds(base, share)], sshare.at[pl.ds(0, share)])
        pltpu.sync_copy(e_sh.at[pl.ds(base, share)], hshare.at[pl.ds(0, share)])
        dshare[pl.ds(share, 16)] = jnp.full((16,), 2 ** 30, _i32)

        po = 0 if is_up else 3
        bias_v = [pvm[po + 0, pl.ds(f * 16, 16)] for f in range(8)]
        g_v = [pvm[po + 1, pl.ds(f * 16, 16)] for f in range(8)]
        be_v = [pvm[po + 2, pl.ds(f * 16, 16)] for f in range(8)]

        hi = lo + CH

        lane = lax.broadcasted_iota(_i32, (16,), 0)

        def scan_body(i, qn):
            d16 = dshare[pl.ds(i * 16, 16)]
            m = (d16 >= lo) & (d16 < hi)
            pos = lane + i * 16
            _, srt = plsc.sort_key_val(jnp.where(m, 0, 1), pos)
            qpos[pl.ds(qn, 16)] = srt
            return qn + jnp.max(lax.cumsum(m.astype(_i32)))

        qn = lax.fori_loop(0, share // 16, scan_body, jnp.int32(0))

        padv = jnp.full((16,), share, _i32)
        for w in range(8):
            qpos[pl.ds(qn + w * 16, 16)] = padv

        nblk = (qn + (K - 1)) // K

        def blk_body(b, carry):
            for f in range(8):
                p16 = qpos[pl.ds(b * K + f * 16, 16)]
                d16 = plsc.load_gather(dshare, [p16])
                s16 = plsc.load_gather(sshare, [p16])
                h16 = plsc.load_gather(hshare, [p16])
                dl = d16 - lo
                ok = (dl >= 0) & (dl < CH)
                blk_dst[pl.ds(f * 16, 16)] = jnp.where(ok, dl, CH)
                blk_src[pl.ds(f * 16, 16)] = jnp.where(ok, s16, 0)
                blk_sh[pl.ds(f * 16, 16)] = jnp.where(ok, h16, 0)
            cp_a = pltpu.async_copy(srcT.at[blk_src], buf_a, sem_a)
            cp_b = pltpu.async_copy(shT.at[blk_sh], buf_b, sem_b)
            cp_a.wait()
            cp_b.wait()

            def row_body(r, c2):
                for f in range(8):
                    a = buf_a[r, pl.ds(f * 16, 16)]
                    bb = buf_b[r, pl.ds(f * 16, 16)]
                    v = jnp.maximum(a + bb + bias_v[f], 0.0)
                    buf_a[r, pl.ds(f * 16, 16)] = v * g_v[f] + be_v[f]
                return c2

            lax.fori_loop(0, K, row_body, 0)
            pltpu.sync_copy(buf_a, acc.at[blk_dst], add=True)
            return carry

        lax.fori_loop(0, nblk, blk_body, 0)

    def do_chunk(t, lo):
        pltpu.sync_copy(xs[t].at[pl.ds(lo + tid * rows_pt, rows_pt)],
                        acc.at[pl.ds(tid * rows_pt, rows_pt)])
        plsc.subcore_barrier()
        for lid, si, hi_, is_up in _TABLE_LISTS[t]:
            run_list(lid, tabs[si], tabs[hi_], is_up, lo)
        plsc.subcore_barrier()
        pltpu.sync_copy(acc.at[pl.ds(tid * rows_pt, rows_pt)],
                        outs[t].at[pl.ds(lo + tid * rows_pt, rows_pt)])
        plsc.subcore_barrier()

    do_chunk(0, cid * CH)

    def c1_body(c, u):
        do_chunk(1, (2 * c + cid) * CH)
        return u

    lax.fori_loop(0, _NP1 // CH // 2, c1_body, 0)

    def c2_body(c, u):
        do_chunk(2, (2 * c + cid) * CH)
        return u

    lax.fori_loop(0, _NP2 // CH // 2, c2_body, 0)


@functools.partial(
    pl.kernel,
    out_type=[jax.ShapeDtypeStruct((_NP0, F), _f32),
              jax.ShapeDtypeStruct((_NP1, F), _f32),
              jax.ShapeDtypeStruct((_NP2, F), _f32)],
    mesh=plsc.VectorSubcoreMesh(core_axis_name="c", subcore_axis_name="s"),
    compiler_params=pltpu.CompilerParams(needs_layout_passes=False),
    scratch_types=[
        pltpu.VMEM((_SHARE_MAX + 16,), _i32),
        pltpu.VMEM((_SHARE_MAX + 16,), _i32),
        pltpu.VMEM((_SHARE_MAX + 16,), _i32),
        pltpu.VMEM((_QCAP,), _i32),
        pltpu.VMEM((K,), _i32),
        pltpu.VMEM((K,), _i32),
        pltpu.VMEM((K,), _i32),
        pltpu.VMEM((K, F), _f32),
        pltpu.VMEM((K, F), _f32),
        pltpu.VMEM((6, F), _f32),
        pltpu.VMEM_SHARED((CH + 8, F), _f32),
        pltpu.SemaphoreType.DMA,
        pltpu.SemaphoreType.DMA,
    ],
)
def _sc_edge(*refs):
    _sc_body(*refs)


def _pad_edges(src, sh, dst):
    e = src.shape[0]
    ep = _rup(e, 256)
    if ep != e:
        src = jnp.pad(src, (0, ep - e))
        sh = jnp.pad(sh, (0, ep - e))
        dst = jnp.pad(dst, (0, ep - e), constant_values=2 ** 30)
    return src.astype(_i32), sh.astype(_i32), dst.astype(_i32)


def kernel(x0, x1, x2, up0_index, up0_shared, up1_index, up1_shared,
           down1_index, down1_shared, down2_index, down2_shared,
           batch0, batch1, batch2, params):
    xp = [_pad_rows(x0, _NP0), _pad_rows(x1, _NP1), _pad_rows(x2, _NP2)]

    s0, h0, d0 = _pad_edges(up0_index[0], up0_shared, up0_index[1])
    s1, h1, d1 = _pad_edges(up1_index[0], up1_shared, up1_index[1])
    s2, h2, d2 = _pad_edges(down1_index[0], down1_shared, down1_index[1])
    s3, h3, d3 = _pad_edges(down2_index[0], down2_shared, down2_index[1])
    edge_args = (s0, h0, d0, s1, h1, d1, s2, h2, d2, s3, h3, d3)

    for l in range(2):
        p = params["layer%d" % l]
        wu_t, wu_b = p["up_W"][:F], p["up_W"][F:]
        wd_t, wd_b = p["down_W"][:F], p["down_W"][F:]
        t0u, t0db = _mm_multi(xp[0], [wu_t, wd_b])
        t1ub, t1ut, t1dt, t1db = _mm_multi(xp[1], [wu_b, wu_t, wd_t, wd_b])
        t2ub, t2dt = _mm_multi(xp[2], [wu_b, wd_t])
        a0, a1, a2 = _sc_edge(
            xp[0], xp[1], xp[2],
            t0u, t1ub, t1ut, t2ub, t1dt, t0db, t2dt, t1db,
            *edge_args,
            p["up_b"], p["up_g"], p["up_be"],
            p["down_b"], p["down_g"], p["down_be"])
        xp = [_upd_mlp(a, p["upd_W1"], p["upd_b1"], p["upd_W2"],
                       p["upd_b2"], p["upd_g"], p["upd_be"])
              for a in (a0, a1, a2)]

    pooled = []
    for xpad, b, n in ((xp[0], batch0, _N0), (xp[1], batch1, _N1),
                       (xp[2], batch2, _N2)):
        bp = jnp.pad(b.astype(_i32), (0, xpad.shape[0] - n),
                     constant_values=NB)
        bp3 = bp.reshape(xpad.shape[0] // 1024, 1, 1024)
        s, c = _pool(xpad, bp3)
        pooled.extend([s, c])

    out = _head(*pooled, params["lin1_W"], params["lin1_b"],
                params["lin2_W"], params["lin2_b"])
    return out[:NB, :NCLS]


# speedup vs baseline: 1.6393x; 1.0073x over previous
import functools

import jax
import jax.numpy as jnp
from jax import lax
from jax.experimental import pallas as pl
from jax.experimental.pallas import tpu as pltpu
from jax.experimental.pallas import tpu_sc as plsc

F = 128
NB = 64
NCLS = 10
CH = 6144
K = 64
SEGP = 128
NEG = -1e30

_f32 = jnp.float32
_i32 = jnp.int32


def _rup(n, m):
    return (n + m - 1) // m * m


def _pad_rows(x, np_):
    n = x.shape[0]
    if n == np_:
        return x
    return jnp.pad(x, ((0, np_ - n), (0, 0)))


def _mm_multi_body(nj, x_ref, w_ref, *out_refs):
    x = x_ref[...]
    for j in range(nj):
        out_refs[j][...] = jnp.dot(x, w_ref[j], preferred_element_type=_f32)


def _mm_multi(x, ws):
    nj = len(ws)
    npad = x.shape[0]
    br = 1024
    w = jnp.stack(ws)
    return pl.pallas_call(
        functools.partial(_mm_multi_body, nj),
        grid=(npad // br,),
        in_specs=[
            pl.BlockSpec((br, F), lambda i: (i, 0)),
            pl.BlockSpec((nj, F, F), lambda i: (0, 0, 0)),
        ],
        out_specs=[pl.BlockSpec((br, F), lambda i: (i, 0))] * nj,
        out_shape=[jax.ShapeDtypeStruct((npad, F), _f32)] * nj,
    )(x, w)


def _upd_body(y_ref, w_ref, v_ref, o_ref):
    y = y_ref[...]
    h = jnp.maximum(jnp.dot(y, w_ref[0], preferred_element_type=_f32)
                    + v_ref[0, :][None, :], 0.0)
    h = jnp.maximum(jnp.dot(h, w_ref[1], preferred_element_type=_f32)
                    + v_ref[1, :][None, :], 0.0)
    o_ref[...] = h * v_ref[2, :][None, :] + v_ref[3, :][None, :]


def _upd_mlp(y, w1, b1, w2, b2, g, be):
    npad = y.shape[0]
    br = 1024
    w = jnp.stack([w1, w2])
    v = jnp.stack([b1, b2, g, be, b1, b1, b1, b1])
    return pl.pallas_call(
        _upd_body,
        grid=(npad // br,),
        in_specs=[
            pl.BlockSpec((br, F), lambda i: (i, 0)),
            pl.BlockSpec((2, F, F), lambda i: (0, 0, 0)),
            pl.BlockSpec((8, F), lambda i: (0, 0)),
        ],
        out_specs=pl.BlockSpec((br, F), lambda i: (i, 0)),
        out_shape=jax.ShapeDtypeStruct((npad, F), _f32),
    )(y, w, v)


def _pool_body(bidx_ref, x_ref, s_ref, c_ref):
    i = pl.program_id(0)
    ids = bidx_ref[...].reshape(1, -1)
    seg = lax.broadcasted_iota(_i32, (SEGP, ids.shape[1]), 0)
    oh = (seg == ids).astype(_f32)
    ps = jnp.dot(oh, x_ref[...], preferred_element_type=_f32)
    pc = jnp.broadcast_to(jnp.sum(oh, axis=1)[:, None], (SEGP, F))

    @pl.when(i == 0)
    def _():
        s_ref[...] = jnp.zeros_like(s_ref)
        c_ref[...] = jnp.zeros_like(c_ref)

    s_ref[...] += ps
    c_ref[...] += pc


def _pool(x, bidx3d):
    npad = x.shape[0]
    br = 1024
    nblk = npad // br
    return pl.pallas_call(
        _pool_body,
        grid=(nblk,),
        in_specs=[
            pl.BlockSpec((1, 1, br), lambda i: (i, 0, 0)),
            pl.BlockSpec((br, F), lambda i: (i, 0)),
        ],
        out_specs=[
            pl.BlockSpec((SEGP, F), lambda i: (0, 0)),
            pl.BlockSpec((SEGP, F), lambda i: (0, 0)),
        ],
        out_shape=[jax.ShapeDtypeStruct((SEGP, F), _f32)] * 2,
    )(bidx3d, x)


def _head_body(s0, c0, s1, c1, s2, c2, w_ref, v_ref, o_ref):
    p = (s0[...] / jnp.maximum(c0[...], 1.0)
         + s1[...] / jnp.maximum(c1[...], 1.0)
         + s2[...] / jnp.maximum(c2[...], 1.0))
    h = jnp.maximum(jnp.dot(p, w_ref[0], preferred_element_type=_f32)
                    + v_ref[0, :][None, :], 0.0)
    lg = jnp.dot(h, w_ref[1], preferred_element_type=_f32) + v_ref[1, :][None, :]
    m = jnp.max(lg, axis=1, keepdims=True)
    lse = m + jnp.log(jnp.sum(jnp.exp(lg - m), axis=1, keepdims=True))
    o_ref[...] = lg - lse


def _head(s0, c0, s1, c1, s2, c2, w1, b1, w2, b2):
    w2p = jnp.pad(w2, ((0, 0), (0, F - NCLS)))
    b2p = jnp.pad(b2, (0, F - NCLS), constant_values=NEG)
    w = jnp.stack([w1, w2p])
    v = jnp.stack([b1, b2p, b1, b1, b1, b1, b1, b1])
    spec = pl.BlockSpec((SEGP, F), lambda: (0, 0))
    return pl.pallas_call(
        _head_body,
        in_specs=[spec] * 6 + [pl.BlockSpec((2, F, F), lambda: (0, 0, 0)),
                               pl.BlockSpec((8, F), lambda: (0, 0))],
        out_specs=spec,
        out_shape=jax.ShapeDtypeStruct((SEGP, F), _f32),
    )(s0, c0, s1, c1, s2, c2, w, v)


_N0, _N1, _N2 = 10000, 80000, 20000
_NP0, _NP1, _NP2 = _rup(_N0, CH), _rup(_N1, CH), _rup(_N2, CH)
_EU0, _EU1, _ED1, _ED2 = 160000, 120000, 160000, 60000
_EP = [_rup(e, 256) for e in (_EU0, _EU1, _ED1, _ED2)]
_SHARES = [e // 16 for e in _EP]
_SHARE_MAX = max(_SHARES)
_QCAP = _SHARE_MAX + 16 + 128

_TASKS = []
for _t, _npad in ((0, _NP0), (1, _NP1), (2, _NP2)):
    for _ci, _lo in enumerate(range(0, _npad, CH)):
        _TASKS.append((_t, _lo, _ci % 2))


_TABLE_LISTS = {
    0: [(0, 3, 4, True)],
    1: [(1, 5, 6, True), (2, 7, 8, False)],
    2: [(3, 9, 10, False)],
}


def _sc_body(*refs):
    (xp0, xp1, xp2,
     t0u, t1ub, t1ut, t2ub, t1dt, t0db, t2dt, t1db,
     e_src0, e_sh0, e_dst0, e_src1, e_sh1, e_dst1,
     e_src2, e_sh2, e_dst2, e_src3, e_sh3, e_dst3,
     upb_h, upg_h, upbe_h, dnb_h, dng_h, dnbe_h,
     out0, out1, out2,
     dshare, sshare, hshare, qpos,
     blk_src0, blk_sh0, blk_dst0, blk_src1, blk_sh1, blk_dst1,
     buf_a0, buf_b0, buf_a1, buf_b1, pvm, acc,
     sem_a0, sem_b0, sem_a1, sem_b1) = refs

    cid = lax.axis_index("c")
    tid = lax.axis_index("s")

    xs = (xp0, xp1, xp2)
    outs = (out0, out1, out2)
    tabs = (None, None, None, t0u, t1ub, t1ut, t2ub, t1dt, t0db, t2dt, t1db)
    elists = ((e_src0, e_sh0, e_dst0), (e_src1, e_sh1, e_dst1),
              (e_src2, e_sh2, e_dst2), (e_src3, e_sh3, e_dst3))

    for j, ph in enumerate((upb_h, upg_h, upbe_h, dnb_h, dng_h, dnbe_h)):
        pltpu.sync_copy(ph, pvm.at[j])

    rows_pt = CH // 16

    def run_list(lid, srcT, shT, is_up, lo):
        share = _SHARES[lid]
        e_src, e_sh, e_dst = elists[lid]
        base = tid * share
        pltpu.sync_copy(e_dst.at[pl.ds(base, share)], dshare.at[pl.ds(0, share)])
        pltpu.sync_copy(e_src.at[pl.ds(base, share)], sshare.at[pl.ds(0, share)])
        pltpu.sync_copy(e_sh.at[pl.ds(base, share)], hshare.at[pl.ds(0, share)])
        dshare[pl.ds(share, 16)] = jnp.full((16,), 2 ** 30, _i32)

        po = 0 if is_up else 3
        bias_v = [pvm[po + 0, pl.ds(f * 16, 16)] for f in range(8)]
        g_v = [pvm[po + 1, pl.ds(f * 16, 16)] for f in range(8)]
        be_v = [pvm[po + 2, pl.ds(f * 16, 16)] for f in range(8)]

        hi = lo + CH

        lane = lax.broadcasted_iota(_i32, (16,), 0)

        def scan_body(i, qn):
            d16 = dshare[pl.ds(i * 16, 16)]
            m = (d16 >= lo) & (d16 < hi)
            pos = lane + i * 16
            _, srt = plsc.sort_key_val(jnp.where(m, 0, 1), pos)
            qpos[pl.ds(qn, 16)] = srt
            return qn + jnp.max(lax.cumsum(m.astype(_i32)))

        qn = lax.fori_loop(0, share // 16, scan_body, jnp.int32(0))

        padv = jnp.full((16,), share, _i32)
        for w in range(8):
            qpos[pl.ds(qn + w * 16, 16)] = padv

        nblk = (qn + (K - 1)) // K

        def prep(b, bsrc, bsh, bdst):
            for f in range(K // 16):
                p16 = qpos[pl.ds(b * K + f * 16, 16)]
                d16 = plsc.load_gather(dshare, [p16])
                s16 = plsc.load_gather(sshare, [p16])
                h16 = plsc.load_gather(hshare, [p16])
                dl = d16 - lo
                ok = (dl >= 0) & (dl < CH)
                bdst[pl.ds(f * 16, 16)] = jnp.where(ok, dl, CH)
                bsrc[pl.ds(f * 16, 16)] = jnp.where(ok, s16, 0)
                bsh[pl.ds(f * 16, 16)] = jnp.where(ok, h16, 0)

        def compute_scatter(ba, bb_, bdst):
            def row_body(r, c2):
                for f in range(8):
                    a = ba[r, pl.ds(f * 16, 16)]
                    bb = bb_[r, pl.ds(f * 16, 16)]
                    v = jnp.maximum(a + bb + bias_v[f], 0.0)
                    ba[r, pl.ds(f * 16, 16)] = v * g_v[f] + be_v[f]
                return c2

            lax.fori_loop(0, K, row_body, 0)
            pltpu.sync_copy(ba, acc.at[bdst], add=True)

        def pair_body(p, carry):
            b0 = 2 * p
            prep(b0, blk_src0, blk_sh0, blk_dst0)
            cpa0 = pltpu.async_copy(srcT.at[blk_src0], buf_a0, sem_a0)
            cpb0 = pltpu.async_copy(shT.at[blk_sh0], buf_b0, sem_b0)
            prep(b0 + 1, blk_src1, blk_sh1, blk_dst1)
            cpa1 = pltpu.async_copy(srcT.at[blk_src1], buf_a1, sem_a1)
            cpb1 = pltpu.async_copy(shT.at[blk_sh1], buf_b1, sem_b1)
            cpa0.wait()
            cpb0.wait()
            compute_scatter(buf_a0, buf_b0, blk_dst0)
            cpa1.wait()
            cpb1.wait()
            compute_scatter(buf_a1, buf_b1, blk_dst1)
            return carry

        lax.fori_loop(0, (nblk + 1) // 2, pair_body, 0)

    def do_chunk(t, lo):
        pltpu.sync_copy(xs[t].at[pl.ds(lo + tid * rows_pt, rows_pt)],
                        acc.at[pl.ds(tid * rows_pt, rows_pt)])
        plsc.subcore_barrier()
        for lid, si, hi_, is_up in _TABLE_LISTS[t]:
            run_list(lid, tabs[si], tabs[hi_], is_up, lo)
        plsc.subcore_barrier()
        pltpu.sync_copy(acc.at[pl.ds(tid * rows_pt, rows_pt)],
                        outs[t].at[pl.ds(lo + tid * rows_pt, rows_pt)])
        plsc.subcore_barrier()

    do_chunk(0, cid * CH)

    def c1_body(c, u):
        do_chunk(1, (2 * c + cid) * CH)
        return u

    lax.fori_loop(0, _NP1 // CH // 2, c1_body, 0)

    def c2_body(c, u):
        do_chunk(2, (2 * c + cid) * CH)
        return u

    lax.fori_loop(0, _NP2 // CH // 2, c2_body, 0)


@functools.partial(
    pl.kernel,
    out_type=[jax.ShapeDtypeStruct((_NP0, F), _f32),
              jax.ShapeDtypeStruct((_NP1, F), _f32),
              jax.ShapeDtypeStruct((_NP2, F), _f32)],
    mesh=plsc.VectorSubcoreMesh(core_axis_name="c", subcore_axis_name="s"),
    compiler_params=pltpu.CompilerParams(needs_layout_passes=False),
    scratch_types=[
        pltpu.VMEM((_SHARE_MAX + 16,), _i32),
        pltpu.VMEM((_SHARE_MAX + 16,), _i32),
        pltpu.VMEM((_SHARE_MAX + 16,), _i32),
        pltpu.VMEM((_QCAP,), _i32),
        pltpu.VMEM((K,), _i32),
        pltpu.VMEM((K,), _i32),
        pltpu.VMEM((K,), _i32),
        pltpu.VMEM((K,), _i32),
        pltpu.VMEM((K,), _i32),
        pltpu.VMEM((K,), _i32),
        pltpu.VMEM((K, F), _f32),
        pltpu.VMEM((K, F), _f32),
        pltpu.VMEM((K, F), _f32),
        pltpu.VMEM((K, F), _f32),
        pltpu.VMEM((6, F), _f32),
        pltpu.VMEM_SHARED((CH + 8, F), _f32),
        pltpu.SemaphoreType.DMA,
        pltpu.SemaphoreType.DMA,
        pltpu.SemaphoreType.DMA,
        pltpu.SemaphoreType.DMA,
    ],
)
def _sc_edge(*refs):
    _sc_body(*refs)


def _pad_edges(src, sh, dst):
    e = src.shape[0]
    ep = _rup(e, 256)
    if ep != e:
        src = jnp.pad(src, (0, ep - e))
        sh = jnp.pad(sh, (0, ep - e))
        dst = jnp.pad(dst, (0, ep - e), constant_values=2 ** 30)
    return src.astype(_i32), sh.astype(_i32), dst.astype(_i32)


def kernel(x0, x1, x2, up0_index, up0_shared, up1_index, up1_shared,
           down1_index, down1_shared, down2_index, down2_shared,
           batch0, batch1, batch2, params):
    xp = [_pad_rows(x0, _NP0), _pad_rows(x1, _NP1), _pad_rows(x2, _NP2)]

    s0, h0, d0 = _pad_edges(up0_index[0], up0_shared, up0_index[1])
    s1, h1, d1 = _pad_edges(up1_index[0], up1_shared, up1_index[1])
    s2, h2, d2 = _pad_edges(down1_index[0], down1_shared, down1_index[1])
    s3, h3, d3 = _pad_edges(down2_index[0], down2_shared, down2_index[1])
    edge_args = (s0, h0, d0, s1, h1, d1, s2, h2, d2, s3, h3, d3)

    for l in range(2):
        p = params["layer%d" % l]
        wu_t, wu_b = p["up_W"][:F], p["up_W"][F:]
        wd_t, wd_b = p["down_W"][:F], p["down_W"][F:]
        t0u, t0db = _mm_multi(xp[0], [wu_t, wd_b])
        t1ub, t1ut, t1dt, t1db = _mm_multi(xp[1], [wu_b, wu_t, wd_t, wd_b])
        t2ub, t2dt = _mm_multi(xp[2], [wu_b, wd_t])
        a0, a1, a2 = _sc_edge(
            xp[0], xp[1], xp[2],
            t0u, t1ub, t1ut, t2ub, t1dt, t0db, t2dt, t1db,
            *edge_args,
            p["up_b"], p["up_g"], p["up_be"],
            p["down_b"], p["down_g"], p["down_be"])
        xp = [_upd_mlp(a, p["upd_W1"], p["upd_b1"], p["upd_W2"],
                       p["upd_b2"], p["upd_g"], p["upd_be"])
              for a in (a0, a1, a2)]

    pooled = []
    for xpad, b, n in ((xp[0], batch0, _N0), (xp[1], batch1, _N1),
                       (xp[2], batch2, _N2)):
        bp = jnp.pad(b.astype(_i32), (0, xpad.shape[0] - n),
                     constant_values=NB)
        bp3 = bp.reshape(xpad.shape[0] // 1024, 1, 1024)
        s, c = _pool(xpad, bp3)
        pooled.extend([s, c])

    out = _head(*pooled, params["lin1_W"], params["lin1_b"],
                params["lin2_W"], params["lin2_b"])
    return out[:NB, :NCLS]


# speedup vs baseline: 1.6573x; 1.0109x over previous
import functools

import jax
import jax.numpy as jnp
from jax import lax
from jax.experimental import pallas as pl
from jax.experimental.pallas import tpu as pltpu
from jax.experimental.pallas import tpu_sc as plsc

F = 128
NB = 64
NCLS = 10
CH = 6144
K = 64
SEGP = 128
NEG = -1e30

_f32 = jnp.float32
_i32 = jnp.int32


def _rup(n, m):
    return (n + m - 1) // m * m


def _pad_rows(x, np_):
    n = x.shape[0]
    if n == np_:
        return x
    return jnp.pad(x, ((0, np_ - n), (0, 0)))


def _mm_multi_body(nj, x_ref, w_ref, b_ref, *out_refs):
    x = x_ref[...]
    for j in range(nj):
        out_refs[j][...] = (jnp.dot(x, w_ref[j], preferred_element_type=_f32)
                            + b_ref[j, :][None, :])


def _mm_multi(x, ws, bs):
    nj = len(ws)
    npad = x.shape[0]
    br = 1024
    w = jnp.stack(ws)
    b = jnp.stack(bs)
    return pl.pallas_call(
        functools.partial(_mm_multi_body, nj),
        grid=(npad // br,),
        in_specs=[
            pl.BlockSpec((br, F), lambda i: (i, 0)),
            pl.BlockSpec((nj, F, F), lambda i: (0, 0, 0)),
            pl.BlockSpec((nj, F), lambda i: (0, 0)),
        ],
        out_specs=[pl.BlockSpec((br, F), lambda i: (i, 0))] * nj,
        out_shape=[jax.ShapeDtypeStruct((npad, F), _f32)] * nj,
    )(x, w, b)


def _upd_body(y_ref, w_ref, v_ref, o_ref):
    y = y_ref[...]
    h = jnp.maximum(jnp.dot(y, w_ref[0], preferred_element_type=_f32)
                    + v_ref[0, :][None, :], 0.0)
    h = jnp.maximum(jnp.dot(h, w_ref[1], preferred_element_type=_f32)
                    + v_ref[1, :][None, :], 0.0)
    o_ref[...] = h * v_ref[2, :][None, :] + v_ref[3, :][None, :]


def _upd_mlp(y, w1, b1, w2, b2, g, be):
    npad = y.shape[0]
    br = 1024
    w = jnp.stack([w1, w2])
    v = jnp.stack([b1, b2, g, be, b1, b1, b1, b1])
    return pl.pallas_call(
        _upd_body,
        grid=(npad // br,),
        in_specs=[
            pl.BlockSpec((br, F), lambda i: (i, 0)),
            pl.BlockSpec((2, F, F), lambda i: (0, 0, 0)),
            pl.BlockSpec((8, F), lambda i: (0, 0)),
        ],
        out_specs=pl.BlockSpec((br, F), lambda i: (i, 0)),
        out_shape=jax.ShapeDtypeStruct((npad, F), _f32),
    )(y, w, v)


def _pool_body(bidx_ref, x_ref, s_ref, c_ref):
    i = pl.program_id(0)
    ids = bidx_ref[...].reshape(1, -1)
    seg = lax.broadcasted_iota(_i32, (SEGP, ids.shape[1]), 0)
    oh = (seg == ids).astype(_f32)
    ps = jnp.dot(oh, x_ref[...], preferred_element_type=_f32)
    pc = jnp.broadcast_to(jnp.sum(oh, axis=1)[:, None], (SEGP, F))

    @pl.when(i == 0)
    def _():
        s_ref[...] = jnp.zeros_like(s_ref)
        c_ref[...] = jnp.zeros_like(c_ref)

    s_ref[...] += ps
    c_ref[...] += pc


def _pool(x, bidx3d):
    npad = x.shape[0]
    br = 1024
    nblk = npad // br
    return pl.pallas_call(
        _pool_body,
        grid=(nblk,),
        in_specs=[
            pl.BlockSpec((1, 1, br), lambda i: (i, 0, 0)),
            pl.BlockSpec((br, F), lambda i: (i, 0)),
        ],
        out_specs=[
            pl.BlockSpec((SEGP, F), lambda i: (0, 0)),
            pl.BlockSpec((SEGP, F), lambda i: (0, 0)),
        ],
        out_shape=[jax.ShapeDtypeStruct((SEGP, F), _f32)] * 2,
    )(bidx3d, x)


def _head_body(s0, c0, s1, c1, s2, c2, w_ref, v_ref, o_ref):
    p = (s0[...] / jnp.maximum(c0[...], 1.0)
         + s1[...] / jnp.maximum(c1[...], 1.0)
         + s2[...] / jnp.maximum(c2[...], 1.0))
    h = jnp.maximum(jnp.dot(p, w_ref[0], preferred_element_type=_f32)
                    + v_ref[0, :][None, :], 0.0)
    lg = jnp.dot(h, w_ref[1], preferred_element_type=_f32) + v_ref[1, :][None, :]
    m = jnp.max(lg, axis=1, keepdims=True)
    lse = m + jnp.log(jnp.sum(jnp.exp(lg - m), axis=1, keepdims=True))
    o_ref[...] = lg - lse


def _head(s0, c0, s1, c1, s2, c2, w1, b1, w2, b2):
    w2p = jnp.pad(w2, ((0, 0), (0, F - NCLS)))
    b2p = jnp.pad(b2, (0, F - NCLS), constant_values=NEG)
    w = jnp.stack([w1, w2p])
    v = jnp.stack([b1, b2p, b1, b1, b1, b1, b1, b1])
    spec = pl.BlockSpec((SEGP, F), lambda: (0, 0))
    return pl.pallas_call(
        _head_body,
        in_specs=[spec] * 6 + [pl.BlockSpec((2, F, F), lambda: (0, 0, 0)),
                               pl.BlockSpec((8, F), lambda: (0, 0))],
        out_specs=spec,
        out_shape=jax.ShapeDtypeStruct((SEGP, F), _f32),
    )(s0, c0, s1, c1, s2, c2, w, v)


_N0, _N1, _N2 = 10000, 80000, 20000
_NP0, _NP1, _NP2 = _rup(_N0, CH), _rup(_N1, CH), _rup(_N2, CH)
_EU0, _EU1, _ED1, _ED2 = 160000, 120000, 160000, 60000
_EP = [_rup(e, 256) for e in (_EU0, _EU1, _ED1, _ED2)]
_SHARES = [e // 16 for e in _EP]
_SHARE_MAX = max(_SHARES)
_QCAP = _SHARE_MAX + 16 + 128

_TASKS = []
for _t, _npad in ((0, _NP0), (1, _NP1), (2, _NP2)):
    for _ci, _lo in enumerate(range(0, _npad, CH)):
        _TASKS.append((_t, _lo, _ci % 2))


_TABLE_LISTS = {
    0: [(0, 3, 4, True)],
    1: [(1, 5, 6, True), (2, 7, 8, False)],
    2: [(3, 9, 10, False)],
}


def _sc_body(*refs):
    (xp0, xp1, xp2,
     t0u, t1ub, t1ut, t2ub, t1dt, t0db, t2dt, t1db,
     e_src0, e_sh0, e_dst0, e_src1, e_sh1, e_dst1,
     e_src2, e_sh2, e_dst2, e_src3, e_sh3, e_dst3,
     upb_h, upg_h, upbe_h, dnb_h, dng_h, dnbe_h,
     out0, out1, out2,
     dshare, sshare, hshare, qpos,
     blk_src0, blk_sh0, blk_dst0, blk_src1, blk_sh1, blk_dst1,
     buf_a0, buf_b0, buf_a1, buf_b1, pvm, acc,
     sem_a0, sem_b0, sem_a1, sem_b1, sem_s0, sem_s1) = refs

    cid = lax.axis_index("c")
    tid = lax.axis_index("s")

    xs = (xp0, xp1, xp2)
    outs = (out0, out1, out2)
    tabs = (None, None, None, t0u, t1ub, t1ut, t2ub, t1dt, t0db, t2dt, t1db)
    elists = ((e_src0, e_sh0, e_dst0), (e_src1, e_sh1, e_dst1),
              (e_src2, e_sh2, e_dst2), (e_src3, e_sh3, e_dst3))

    for j, ph in enumerate((upb_h, upg_h, upbe_h, dnb_h, dng_h, dnbe_h)):
        pltpu.sync_copy(ph, pvm.at[j])

    rows_pt = CH // 16

    def run_list(lid, srcT, shT, is_up, lo):
        share = _SHARES[lid]
        e_src, e_sh, e_dst = elists[lid]
        base = tid * share
        pltpu.sync_copy(e_dst.at[pl.ds(base, share)], dshare.at[pl.ds(0, share)])
        pltpu.sync_copy(e_src.at[pl.ds(base, share)], sshare.at[pl.ds(0, share)])
        pltpu.sync_copy(e_sh.at[pl.ds(base, share)], hshare.at[pl.ds(0, share)])
        dshare[pl.ds(share, 16)] = jnp.full((16,), 2 ** 30, _i32)

        po = 0 if is_up else 3
        g_v = [pvm[po + 1, pl.ds(f * 16, 16)] for f in range(8)]
        be_v = [pvm[po + 2, pl.ds(f * 16, 16)] for f in range(8)]

        hi = lo + CH

        lane = lax.broadcasted_iota(_i32, (16,), 0)

        def scan_body(i, qn):
            d16 = dshare[pl.ds(i * 16, 16)]
            m = (d16 >= lo) & (d16 < hi)
            pos = lane + i * 16
            _, srt = plsc.sort_key_val(jnp.where(m, 0, 1), pos)
            qpos[pl.ds(qn, 16)] = srt
            return qn + jnp.max(lax.cumsum(m.astype(_i32)))

        qn = lax.fori_loop(0, share // 16, scan_body, jnp.int32(0))

        padv = jnp.full((16,), share, _i32)
        for w in range(8):
            qpos[pl.ds(qn + w * 16, 16)] = padv

        nblk = (qn + (K - 1)) // K

        def prep(b, bsrc, bsh, bdst):
            for f in range(K // 16):
                p16 = qpos[pl.ds(b * K + f * 16, 16)]
                d16 = plsc.load_gather(dshare, [p16])
                s16 = plsc.load_gather(sshare, [p16])
                h16 = plsc.load_gather(hshare, [p16])
                dl = d16 - lo
                ok = (dl >= 0) & (dl < CH)
                bdst[pl.ds(f * 16, 16)] = jnp.where(ok, dl, CH)
                bsrc[pl.ds(f * 16, 16)] = jnp.where(ok, s16, 0)
                bsh[pl.ds(f * 16, 16)] = jnp.where(ok, h16, 0)

        def compute(ba, bb_):
            def row_body(r, c2):
                for f in range(8):
                    a = ba[r, pl.ds(f * 16, 16)]
                    bb = bb_[r, pl.ds(f * 16, 16)]
                    v = jnp.maximum(a + bb, 0.0)
                    ba[r, pl.ds(f * 16, 16)] = v * g_v[f] + be_v[f]
                return c2

            lax.fori_loop(0, K, row_body, 0)

        def pair_body(p, carry):
            @pl.when(p > 0)
            def _():
                pltpu.make_async_copy(buf_a0, acc.at[blk_dst0], sem_s0).wait()
                pltpu.make_async_copy(buf_a1, acc.at[blk_dst1], sem_s1).wait()

            b0 = 2 * p
            prep(b0, blk_src0, blk_sh0, blk_dst0)
            cpa0 = pltpu.async_copy(srcT.at[blk_src0], buf_a0, sem_a0)
            cpb0 = pltpu.async_copy(shT.at[blk_sh0], buf_b0, sem_b0)
            prep(b0 + 1, blk_src1, blk_sh1, blk_dst1)
            cpa1 = pltpu.async_copy(srcT.at[blk_src1], buf_a1, sem_a1)
            cpb1 = pltpu.async_copy(shT.at[blk_sh1], buf_b1, sem_b1)
            cpa0.wait()
            cpb0.wait()
            compute(buf_a0, buf_b0)
            pltpu.async_copy(buf_a0, acc.at[blk_dst0], sem_s0, add=True)
            cpa1.wait()
            cpb1.wait()
            compute(buf_a1, buf_b1)
            pltpu.async_copy(buf_a1, acc.at[blk_dst1], sem_s1, add=True)
            return carry

        lax.fori_loop(0, (nblk + 1) // 2, pair_body, 0)

        @pl.when(nblk > 0)
        def _():
            pltpu.make_async_copy(buf_a0, acc.at[blk_dst0], sem_s0).wait()
            pltpu.make_async_copy(buf_a1, acc.at[blk_dst1], sem_s1).wait()

    def do_chunk(t, lo):
        pltpu.sync_copy(xs[t].at[pl.ds(lo + tid * rows_pt, rows_pt)],
                        acc.at[pl.ds(tid * rows_pt, rows_pt)])
        plsc.subcore_barrier()
        for lid, si, hi_, is_up in _TABLE_LISTS[t]:
            run_list(lid, tabs[si], tabs[hi_], is_up, lo)
        plsc.subcore_barrier()
        pltpu.sync_copy(acc.at[pl.ds(tid * rows_pt, rows_pt)],
                        outs[t].at[pl.ds(lo + tid * rows_pt, rows_pt)])
        plsc.subcore_barrier()

    do_chunk(0, cid * CH)

    def c1_body(c, u):
        do_chunk(1, (2 * c + cid) * CH)
        return u

    lax.fori_loop(0, _NP1 // CH // 2, c1_body, 0)

    def c2_body(c, u):
        do_chunk(2, (2 * c + cid) * CH)
        return u

    lax.fori_loop(0, _NP2 // CH // 2, c2_body, 0)


@functools.partial(
    pl.kernel,
    out_type=[jax.ShapeDtypeStruct((_NP0, F), _f32),
              jax.ShapeDtypeStruct((_NP1, F), _f32),
              jax.ShapeDtypeStruct((_NP2, F), _f32)],
    mesh=plsc.VectorSubcoreMesh(core_axis_name="c", subcore_axis_name="s"),
    compiler_params=pltpu.CompilerParams(needs_layout_passes=False),
    scratch_types=[
        pltpu.VMEM((_SHARE_MAX + 16,), _i32),
        pltpu.VMEM((_SHARE_MAX + 16,), _i32),
        pltpu.VMEM((_SHARE_MAX + 16,), _i32),
        pltpu.VMEM((_QCAP,), _i32),
        pltpu.VMEM((K,), _i32),
        pltpu.VMEM((K,), _i32),
        pltpu.VMEM((K,), _i32),
        pltpu.VMEM((K,), _i32),
        pltpu.VMEM((K,), _i32),
        pltpu.VMEM((K,), _i32),
        pltpu.VMEM((K, F), _f32),
        pltpu.VMEM((K, F), _f32),
        pltpu.VMEM((K, F), _f32),
        pltpu.VMEM((K, F), _f32),
        pltpu.VMEM((6, F), _f32),
        pltpu.VMEM_SHARED((CH + 8, F), _f32),
        pltpu.SemaphoreType.DMA,
        pltpu.SemaphoreType.DMA,
        pltpu.SemaphoreType.DMA,
        pltpu.SemaphoreType.DMA,
        pltpu.SemaphoreType.DMA,
        pltpu.SemaphoreType.DMA,
    ],
)
def _sc_edge(*refs):
    _sc_body(*refs)


def _pad_edges(src, sh, dst):
    e = src.shape[0]
    ep = _rup(e, 256)
    if ep != e:
        src = jnp.pad(src, (0, ep - e))
        sh = jnp.pad(sh, (0, ep - e))
        dst = jnp.pad(dst, (0, ep - e), constant_values=2 ** 30)
    return src.astype(_i32), sh.astype(_i32), dst.astype(_i32)


def kernel(x0, x1, x2, up0_index, up0_shared, up1_index, up1_shared,
           down1_index, down1_shared, down2_index, down2_shared,
           batch0, batch1, batch2, params):
    xp = [_pad_rows(x0, _NP0), _pad_rows(x1, _NP1), _pad_rows(x2, _NP2)]

    s0, h0, d0 = _pad_edges(up0_index[0], up0_shared, up0_index[1])
    s1, h1, d1 = _pad_edges(up1_index[0], up1_shared, up1_index[1])
    s2, h2, d2 = _pad_edges(down1_index[0], down1_shared, down1_index[1])
    s3, h3, d3 = _pad_edges(down2_index[0], down2_shared, down2_index[1])
    edge_args = (s0, h0, d0, s1, h1, d1, s2, h2, d2, s3, h3, d3)

    for l in range(2):
        p = params["layer%d" % l]
        wu_t, wu_b = p["up_W"][:F], p["up_W"][F:]
        wd_t, wd_b = p["down_W"][:F], p["down_W"][F:]
        z = jnp.zeros((F,), _f32)
        ub, db = p["up_b"], p["down_b"]
        t0u, t0db = _mm_multi(xp[0], [wu_t, wd_b], [ub, z])
        t1ub, t1ut, t1dt, t1db = _mm_multi(
            xp[1], [wu_b, wu_t, wd_t, wd_b], [z, ub, db, z])
        t2ub, t2dt = _mm_multi(xp[2], [wu_b, wd_t], [z, db])
        a0, a1, a2 = _sc_edge(
            xp[0], xp[1], xp[2],
            t0u, t1ub, t1ut, t2ub, t1dt, t0db, t2dt, t1db,
            *edge_args,
            p["up_b"], p["up_g"], p["up_be"],
            p["down_b"], p["down_g"], p["down_be"])
        xp = [_upd_mlp(a, p["upd_W1"], p["upd_b1"], p["upd_W2"],
                       p["upd_b2"], p["upd_g"], p["upd_be"])
              for a in (a0, a1, a2)]

    pooled = []
    for xpad, b, n in ((xp[0], batch0, _N0), (xp[1], batch1, _N1),
                       (xp[2], batch2, _N2)):
        bp = jnp.pad(b.astype(_i32), (0, xpad.shape[0] - n),
                     constant_values=NB)
        bp3 = bp.reshape(xpad.shape[0] // 1024, 1, 1024)
        s, c = _pool(xpad, bp3)
        pooled.extend([s, c])

    out = _head(*pooled, params["lin1_W"], params["lin1_b"],
                params["lin2_W"], params["lin2_b"])
    return out[:NB, :NCLS]
